# baseline (speedup 1.0000x reference)
import functools

import jax
import jax.numpy as jnp
from jax import lax
from jax.experimental import pallas as pl
from jax.experimental.pallas import tpu as pltpu
from jax.experimental.pallas import tpu_sc as plsc

N = 128
K = 16
H = 16
OUT = 64
TEMP = 0.5
EPS = 1e-08
GB = 16
G_TOTAL = 128
NW = 32


def _bf(v):
    return jax.lax.convert_element_type(
        jax.lax.convert_element_type(v, jnp.bfloat16), jnp.float32)


def _graph_kernel(u_ref, x_ref, xcol_ref, phi_ref, wcat_ref, bcat_ref, a_ref,
                  wn_ref, bn_ref, A_ref, norms_ref, emb_ref):
    R = GB * N
    us = u_ref[...].reshape(R, N)
    phi = phi_ref[...]

    gum = -jnp.log(-jnp.log(us + 1e-09) + 1e-09)
    s = (phi + gum) * (1.0 / TEMP)
    rmax = jnp.max(s, axis=1, keepdims=True)
    e = jnp.exp(s - rmax)
    rsum = jnp.sum(e, axis=1, keepdims=True)
    A = e / rsum
    A_ref[...] = A.reshape(GB, N, N)
    logA = jnp.log(A + EPS)

    ones_row = jnp.ones((1, N), jnp.float32)

    accs = []
    for g in range(GB):
        xr = x_ref[g]
        xi_b = jax.lax.dot_general(xr, ones_row, (((0,), (0,)), ((), ())),
                                   preferred_element_type=jnp.float32)
        xj_b = _bf(xr)
        acc = jnp.zeros((N, N), jnp.float32)
        for h in range(H):
            w1h = _bf(wcat_ref[0, h])
            w2h = _bf(wcat_ref[1, h])
            bh = bcat_ref[0, h]
            ah = _bf(a_ref[0, h])
            q = xj_b * w2h + bh
            t = xi_b * w1h + q
            lr = jnp.maximum(t, 0.2 * t)
            acc = acc + _bf(lr) * ah
        accs.append(acc)

    scores = jnp.concatenate(accs, axis=0) + logA

    rmax2 = jnp.max(scores, axis=1, keepdims=True)
    ex = jnp.exp(scores - rmax2)
    alpha = ex / jnp.sum(ex, axis=1, keepdims=True)
    alpha_b = jax.lax.convert_element_type(alpha, jnp.bfloat16)

    wn = wn_ref[...]
    bn = bn_ref[...]

    embs = []
    norms = []
    for g in range(GB):
        np_mat = xcol_ref[g] * wn + bn
        npb = jax.lax.convert_element_type(np_mat, jnp.bfloat16)
        emb = jax.lax.dot_general(
            alpha_b[g * N:(g + 1) * N], npb,
            (((1,), (0,)), ((), ())),
            preferred_element_type=jnp.float32)
        emb = jnp.maximum(emb, 0.0)
        nsq = jnp.sum(emb * emb, axis=1, keepdims=True)
        norms.append(jnp.sqrt(nsq)[None])
        embs.append(emb[None])

    norms_ref[...] = jnp.concatenate(norms, axis=0)
    emb_ref[...] = jnp.concatenate(embs, axis=0)


def _lt(ak, ai, bk, bi):
    return (ak > bk) | ((ak == bk) & (ai < bi))


def _take(v, p):
    return lax.gather(
        v, p[:, None],
        lax.GatherDimensionNumbers(offset_dims=(), collapsed_slice_dims=(0,),
                                   start_index_map=(0,)),
        (1,), mode=lax.GatherScatterMode.PROMISE_IN_BOUNDS)


def _bitonic_stages(keys, idxs, lane, stages):
    for k_, j_ in stages:
        p = lane ^ j_
        ok = _take(keys, p)
        oi = _take(idxs, p)
        lower = (lane & j_) == 0
        updir = (lane & k_) == 0
        c = _lt(keys, idxs, ok, oi)
        take_cur = lower ^ updir ^ c
        keys = jnp.where(take_cur, keys, ok)
        idxs = jnp.where(take_cur, idxs, oi)
    return keys, idxs


_SORT16 = [(2, 1), (4, 2), (4, 1), (8, 4), (8, 2), (8, 1),
           (16, 8), (16, 4), (16, 2), (16, 1)]
_MERGE16 = [(16, 8), (16, 4), (16, 2), (16, 1)]


def _merge_top(ak, ai, bk, bi, lane):
    rbk = lax.rev(bk, (0,))
    rbi = lax.rev(bi, (0,))
    c = _lt(ak, ai, rbk, rbi)
    tk = jnp.where(c, ak, rbk)
    ti = jnp.where(c, ai, rbi)
    return _bitonic_stages(tk, ti, lane, _MERGE16)


def _sc_body(norms_hbm, emb_hbm, idx_hbm, esel_hbm,
             norms_v, gidx_v, idx_v, grp_v, esel_v, sem):
    wid = lax.axis_index("s") * 2 + lax.axis_index("c")
    lane = lax.iota(jnp.int32, 16)
    for t in range(G_TOTAL // NW):
        g = wid + NW * t
        pltpu.sync_copy(norms_hbm.at[g], norms_v)

        chunks = []
        for c in range(8):
            keyc = norms_v[pl.ds(c * 16, 16)]
            idxc = lane + c * 16
            chunks.append(_bitonic_stages(keyc, idxc, lane, _SORT16))
        while len(chunks) > 1:
            nxt = []
            for a in range(0, len(chunks), 2):
                (ak, ai), (bk, bi) = chunks[a], chunks[a + 1]
                nxt.append(_merge_top(ak, ai, bk, bi, lane))
            chunks = nxt
        top_k, top_i = chunks[0]

        idx_v[...] = top_i
        gidx_v[...] = lax.shift_right_logical(top_i + g * N, 3)
        pltpu.async_copy(emb_hbm.at[gidx_v], grp_v, sem).wait()
        for k in range(K):
            off = (top_i[k] & 7) * H
            row = grp_v[pl.ds(k, 1), pl.ds(off, H)].reshape(H)
            esel_v[pl.ds(k * H, H)] = row

        pltpu.sync_copy(idx_v, idx_hbm.at[g])
        pltpu.sync_copy(esel_v, esel_hbm.at[g])


def _sc_topk(norms3, emb3):
    mesh = plsc.VectorSubcoreMesh(core_axis_name="c", subcore_axis_name="s")
    kern = functools.partial(
        pl.kernel,
        mesh=mesh,
        out_type=[
            jax.ShapeDtypeStruct((G_TOTAL, K), jnp.int32),
            jax.ShapeDtypeStruct((G_TOTAL, K * H), jnp.float32),
        ],
        scratch_types=[
            pltpu.VMEM((N,), jnp.float32),
            pltpu.VMEM((K,), jnp.int32),
            pltpu.VMEM((K,), jnp.int32),
            pltpu.VMEM((K, 128), jnp.float32),
            pltpu.VMEM((K * H,), jnp.float32),
            pltpu.SemaphoreType.DMA,
        ],
    )(_sc_body)
    return kern(norms3, emb3)


def _proj_kernel(x_ref, w_ref, b_ref, o_ref):
    o_ref[...] = jax.lax.dot_general(
        x_ref[...], w_ref[...], (((1,), (0,)), ((), ())),
        preferred_element_type=jnp.float32) + b_ref[...]


@jax.jit
def kernel(x, u, phi, W_cat_w, W_cat_b, a_w, W_node_w, W_node_b, proj_w, proj_b):
    B, order, n = x.shape
    G = B * order

    x2 = x.reshape(G, 1, n)
    x3 = x.reshape(G, n, 1)
    phi_t = jnp.tile(phi, (GB, 1))
    bcat = W_cat_b.reshape(1, H)
    a_row = a_w.reshape(1, H)
    wn = W_node_w.reshape(1, H)
    bn = W_node_b.reshape(1, H)

    A_out, norms3, emb3 = pl.pallas_call(
        _graph_kernel,
        grid=(G // GB,),
        in_specs=[
            pl.BlockSpec((GB, N, N), lambda g: (g, 0, 0)),
            pl.BlockSpec((GB, 1, N), lambda g: (g, 0, 0)),
            pl.BlockSpec((GB, N, 1), lambda g: (g, 0, 0)),
            pl.BlockSpec((GB * N, N), lambda g: (0, 0)),
            pl.BlockSpec(memory_space=pltpu.SMEM),
            pl.BlockSpec(memory_space=pltpu.SMEM),
            pl.BlockSpec(memory_space=pltpu.SMEM),
            pl.BlockSpec((1, H), lambda g: (0, 0)),
            pl.BlockSpec((1, H), lambda g: (0, 0)),
        ],
        out_specs=[
            pl.BlockSpec((GB, N, N), lambda g: (g, 0, 0)),
            pl.BlockSpec((GB, N, 1), lambda g: (g, 0, 0)),
            pl.BlockSpec((GB, N, H), lambda g: (g, 0, 0)),
        ],
        out_shape=[
            jax.ShapeDtypeStruct((G, N, N), jnp.float32),
            jax.ShapeDtypeStruct((G, N, 1), jnp.float32),
            jax.ShapeDtypeStruct((G, N, H), jnp.float32),
        ],
        compiler_params=pltpu.CompilerParams(
            dimension_semantics=("arbitrary",),
        ),
    )(u, x2, x3, phi_t, W_cat_w, bcat, a_row, wn, bn)

    idx_out, esel_out = _sc_topk(norms3.reshape(G, n), emb3.reshape(G * n * H // 128, 128))

    sel_flat = esel_out
    projected = pl.pallas_call(
        _proj_kernel,
        in_specs=[
            pl.BlockSpec((G, K * H), lambda: (0, 0)),
            pl.BlockSpec((K * H, OUT), lambda: (0, 0)),
            pl.BlockSpec((1, OUT), lambda: (0, 0)),
        ],
        out_specs=pl.BlockSpec((G, OUT), lambda: (0, 0)),
        out_shape=jax.ShapeDtypeStruct((G, OUT), jnp.float32),
    )(sel_flat, proj_w, proj_b.reshape(1, OUT))

    return (projected.reshape(B, order, OUT),
            idx_out.reshape(B, order, K),
            A_out.reshape(B, order, n, n))


# speedup vs baseline: 1.0741x; 1.0741x over previous
import functools

import jax
import jax.numpy as jnp
from jax import lax
from jax.experimental import pallas as pl
from jax.experimental.pallas import tpu as pltpu
from jax.experimental.pallas import tpu_sc as plsc

N = 128
K = 16
H = 16
OUT = 64
TEMP = 0.5
EPS = 1e-08
GB = 16
G_TOTAL = 128
NW = 32


def _bf(v):
    return jax.lax.convert_element_type(
        jax.lax.convert_element_type(v, jnp.bfloat16), jnp.float32)


def _graph_kernel(u_ref, x_ref, xcol_ref, phi_ref, wcat_ref, bcat_ref, a_ref,
                  wn_ref, bn_ref, A_ref, norms_ref, emb_ref):
    R = GB * N
    us = u_ref[...].reshape(R, N)
    phi = phi_ref[...]

    gum = -jnp.log(-jnp.log(us + 1e-09) + 1e-09)
    s = (phi + gum) * (1.0 / TEMP)
    rmax = jnp.max(s, axis=1, keepdims=True)
    e = jnp.exp(s - rmax)
    rsum = jnp.sum(e, axis=1, keepdims=True)
    A = e / rsum
    A_ref[...] = A.reshape(GB, N, N)
    logA = jnp.log(A + EPS)

    ones_row = jnp.ones((1, N), jnp.float32)

    accs = []
    for g in range(GB):
        xr = x_ref[g]
        xi_b = jax.lax.dot_general(xr, ones_row, (((0,), (0,)), ((), ())),
                                   preferred_element_type=jnp.float32)
        xj_b = _bf(xr)
        acc = jnp.zeros((N, N), jnp.float32)
        for h in range(H):
            w1h = _bf(wcat_ref[0, h])
            w2h = _bf(wcat_ref[1, h])
            bh = bcat_ref[0, h]
            ah = _bf(a_ref[0, h])
            q = xj_b * w2h + bh
            t = xi_b * w1h + q
            lr = jnp.maximum(t, 0.2 * t)
            acc = acc + _bf(lr) * ah
        accs.append(acc)

    scores = jnp.concatenate(accs, axis=0) + logA

    rmax2 = jnp.max(scores, axis=1, keepdims=True)
    ex = jnp.exp(scores - rmax2)
    alpha = ex / jnp.sum(ex, axis=1, keepdims=True)
    alpha_b = jax.lax.convert_element_type(alpha, jnp.bfloat16)

    wn = wn_ref[...]
    bn = bn_ref[...]

    embs = []
    norms = []
    zpad = jnp.zeros((1, 128 - H), jnp.float32)
    for g in range(GB):
        np_mat = xcol_ref[g] * wn + bn
        npb = jax.lax.convert_element_type(np_mat, jnp.bfloat16)
        emb = jax.lax.dot_general(
            alpha_b[g * N:(g + 1) * N], npb,
            (((1,), (0,)), ((), ())),
            preferred_element_type=jnp.float32)
        emb = jnp.maximum(emb, 0.0)
        nsq = jnp.sum(emb * emb, axis=1, keepdims=True)
        norms.append(jnp.sqrt(nsq)[None])
        np_pad = jnp.concatenate([np_mat, jnp.broadcast_to(zpad, (N, 128 - H))],
                                 axis=1)
        emb_pad = jax.lax.dot_general(
            alpha_b[g * N:(g + 1) * N],
            jax.lax.convert_element_type(np_pad, jnp.bfloat16),
            (((1,), (0,)), ((), ())),
            preferred_element_type=jnp.float32)
        embs.append(jnp.maximum(emb_pad, 0.0)[None])

    norms_ref[...] = jnp.concatenate(norms, axis=0)
    emb_ref[...] = jnp.concatenate(embs, axis=0)


def _lt(ak, ai, bk, bi):
    return (ak > bk) | ((ak == bk) & (ai < bi))


def _take(v, p):
    return lax.gather(
        v, p[:, None],
        lax.GatherDimensionNumbers(offset_dims=(), collapsed_slice_dims=(0,),
                                   start_index_map=(0,)),
        (1,), mode=lax.GatherScatterMode.PROMISE_IN_BOUNDS)


def _bitonic_stages(keys, idxs, lane, stages):
    for k_, j_ in stages:
        p = lane ^ j_
        ok = _take(keys, p)
        oi = _take(idxs, p)
        lower = (lane & j_) == 0
        updir = (lane & k_) == 0
        c = _lt(keys, idxs, ok, oi)
        take_cur = lower ^ updir ^ c
        keys = jnp.where(take_cur, keys, ok)
        idxs = jnp.where(take_cur, idxs, oi)
    return keys, idxs


_SORT16 = [(2, 1), (4, 2), (4, 1), (8, 4), (8, 2), (8, 1),
           (16, 8), (16, 4), (16, 2), (16, 1)]
_MERGE16 = [(16, 8), (16, 4), (16, 2), (16, 1)]


def _merge_top(ak, ai, bk, bi, lane):
    rbk = lax.rev(bk, (0,))
    rbi = lax.rev(bi, (0,))
    c = _lt(ak, ai, rbk, rbi)
    tk = jnp.where(c, ak, rbk)
    ti = jnp.where(c, ai, rbi)
    return _bitonic_stages(tk, ti, lane, _MERGE16)


def _sc_body(norms_hbm, emb_hbm, idx_hbm, esel_hbm,
             norms_v, idx_v, esel_v, sem_n, sem_g, sem_o):
    wid = lax.axis_index("s") * 2 + lax.axis_index("c")
    lane = lax.iota(jnp.int32, 16)
    T = G_TOTAL // NW
    gs = [wid + NW * t for t in range(T)]

    nh = [pltpu.async_copy(norms_hbm.at[gs[t]], norms_v.at[t], sem_n)
          for t in range(T)]
    gh = []
    for t in range(T):
        nh[t].wait()
        chunks = []
        nv = norms_v.at[t]
        for c in range(8):
            keyc = nv[pl.ds(c * 16, 16)]
            idxc = lane + c * 16
            chunks.append(_bitonic_stages(keyc, idxc, lane, _SORT16))
        while len(chunks) > 1:
            nxt = []
            for a in range(0, len(chunks), 2):
                (ak, ai), (bk, bi) = chunks[a], chunks[a + 1]
                nxt.append(_merge_top(ak, ai, bk, bi, lane))
            chunks = nxt
        top_k, top_i = chunks[0]

        ivt = idx_v.at[t]
        ivt[...] = top_i
        gh.append(pltpu.async_copy(emb_hbm.at[top_i + gs[t] * N],
                                   esel_v.at[t], sem_g))
    oh = []
    for t in range(T):
        gh[t].wait()
        oh.append(pltpu.async_copy(idx_v.at[t], idx_hbm.at[gs[t]], sem_o))
        oh.append(pltpu.async_copy(esel_v.at[t], esel_hbm.at[gs[t]], sem_o))
    for h in oh:
        h.wait()


def _sc_topk(norms3, emb3):
    mesh = plsc.VectorSubcoreMesh(core_axis_name="c", subcore_axis_name="s")
    kern = functools.partial(
        pl.kernel,
        mesh=mesh,
        out_type=[
            jax.ShapeDtypeStruct((G_TOTAL, K), jnp.int32),
            jax.ShapeDtypeStruct((G_TOTAL, K, 128), jnp.float32),
        ],
        scratch_types=[
            pltpu.VMEM((G_TOTAL // NW, N), jnp.float32),
            pltpu.VMEM((G_TOTAL // NW, K), jnp.int32),
            pltpu.VMEM((G_TOTAL // NW, K, 128), jnp.float32),
            pltpu.SemaphoreType.DMA,
            pltpu.SemaphoreType.DMA,
            pltpu.SemaphoreType.DMA,
        ],
    )(_sc_body)
    return kern(norms3, emb3)


def _proj_kernel(x_ref, w_ref, b_ref, o_ref):
    o_ref[...] = jax.lax.dot_general(
        x_ref[...], w_ref[...], (((1,), (0,)), ((), ())),
        preferred_element_type=jnp.float32) + b_ref[...]


@jax.jit
def kernel(x, u, phi, W_cat_w, W_cat_b, a_w, W_node_w, W_node_b, proj_w, proj_b):
    B, order, n = x.shape
    G = B * order

    x2 = x.reshape(G, 1, n)
    x3 = x.reshape(G, n, 1)
    phi_t = jnp.tile(phi, (GB, 1))
    bcat = W_cat_b.reshape(1, H)
    a_row = a_w.reshape(1, H)
    wn = W_node_w.reshape(1, H)
    bn = W_node_b.reshape(1, H)

    A_out, norms3, emb3 = pl.pallas_call(
        _graph_kernel,
        grid=(G // GB,),
        in_specs=[
            pl.BlockSpec((GB, N, N), lambda g: (g, 0, 0)),
            pl.BlockSpec((GB, 1, N), lambda g: (g, 0, 0)),
            pl.BlockSpec((GB, N, 1), lambda g: (g, 0, 0)),
            pl.BlockSpec((GB * N, N), lambda g: (0, 0)),
            pl.BlockSpec(memory_space=pltpu.SMEM),
            pl.BlockSpec(memory_space=pltpu.SMEM),
            pl.BlockSpec(memory_space=pltpu.SMEM),
            pl.BlockSpec((1, H), lambda g: (0, 0)),
            pl.BlockSpec((1, H), lambda g: (0, 0)),
        ],
        out_specs=[
            pl.BlockSpec((GB, N, N), lambda g: (g, 0, 0)),
            pl.BlockSpec((GB, N, 1), lambda g: (g, 0, 0)),
            pl.BlockSpec((GB, N, 128), lambda g: (g, 0, 0)),
        ],
        out_shape=[
            jax.ShapeDtypeStruct((G, N, N), jnp.float32),
            jax.ShapeDtypeStruct((G, N, 1), jnp.float32),
            jax.ShapeDtypeStruct((G, N, 128), jnp.float32),
        ],
        compiler_params=pltpu.CompilerParams(
            dimension_semantics=("arbitrary",),
        ),
    )(u, x2, x3, phi_t, W_cat_w, bcat, a_row, wn, bn)

    idx_out, esel_out = _sc_topk(norms3.reshape(G, n), emb3.reshape(G * n, 128))

    sel_flat = esel_out[:, :, :H].reshape(G, K * H)
    projected = pl.pallas_call(
        _proj_kernel,
        in_specs=[
            pl.BlockSpec((G, K * H), lambda: (0, 0)),
            pl.BlockSpec((K * H, OUT), lambda: (0, 0)),
            pl.BlockSpec((1, OUT), lambda: (0, 0)),
        ],
        out_specs=pl.BlockSpec((G, OUT), lambda: (0, 0)),
        out_shape=jax.ShapeDtypeStruct((G, OUT), jnp.float32),
    )(sel_flat, proj_w, proj_b.reshape(1, OUT))

    return (projected.reshape(B, order, OUT),
            idx_out.reshape(B, order, K),
            A_out.reshape(B, order, n, n))
